# baseline (speedup 1.0000x reference)
import functools

import jax
import jax.numpy as jnp
from jax import lax
from jax.experimental import pallas as pl
from jax.experimental.pallas import tpu as pltpu
from jax.experimental.pallas import tpu_sc as plsc

N = 10000
M = 32
F = 42
FP = 48
D3 = 3
FD = F * D3
H1, H2 = 64, 32
FE = 128
NPAD = 10016
BLK = 1024
B2 = N * M
NWK = 32
PW = B2 // NWK
CH = 400
NCH = PW // CH
BA = 400
BAM = BA * M


def _mlp_body(img_ref, w1_ref, b1_ref, w2_ref, b2_ref, w3t_ref, b3_ref,
              ei_ref, de_ref, etot_ref):
    i = pl.program_id(0)
    img = img_ref[...]
    z1 = jnp.dot(img, w1_ref[...],
                 preferred_element_type=jnp.float32) + b1_ref[...]
    a1 = jax.nn.sigmoid(z1)
    z2 = jnp.dot(a1, w2_ref[...],
                 preferred_element_type=jnp.float32) + b2_ref[...]
    a2 = jax.nn.sigmoid(z2)
    w3t = w3t_ref[...]
    ei = jnp.sum(a2 * w3t, axis=1, keepdims=True) + b3_ref[...]
    rid = i * BLK + lax.broadcasted_iota(jnp.int32, (BLK, 1), 0)
    valid = rid < N
    ei = jnp.where(valid, ei, 0.0)
    g2 = a2 * (1.0 - a2) * w3t
    g1 = lax.dot_general(g2, w2_ref[...], (((1,), (1,)), ((), ())),
                         preferred_element_type=jnp.float32)
    g1 = g1 * (a1 * (1.0 - a1))
    de = lax.dot_general(g1, w1_ref[...], (((1,), (1,)), ((), ())),
                         preferred_element_type=jnp.float32)
    de = jnp.where(valid, de, 0.0)
    ei_ref[...] = ei
    de_ref[...] = de

    @pl.when(i == 0)
    def _init():
        etot_ref[...] = jnp.zeros_like(etot_ref)

    etot_ref[...] += jnp.sum(ei)


def _run_mlp(img_pad, w1p, b1r, w2, b2r, w3t, b3r):
    grid = (NPAD + BLK - 1) // BLK
    full = lambda s: pl.BlockSpec(s, lambda i: tuple(0 for _ in s))
    return pl.pallas_call(
        _mlp_body,
        grid=(grid,),
        in_specs=[
            pl.BlockSpec((BLK, FP), lambda i: (i, 0)),
            full((FP, H1)), full((1, H1)),
            full((H1, H2)), full((1, H2)),
            full((1, H2)), full((1, 1)),
        ],
        out_specs=[
            pl.BlockSpec((BLK, 1), lambda i: (i, 0)),
            pl.BlockSpec((BLK, FP), lambda i: (i, 0)),
            pl.BlockSpec((1, 1), lambda i: (0, 0)),
        ],
        out_shape=[
            jax.ShapeDtypeStruct((NPAD, 1), jnp.float32),
            jax.ShapeDtypeStruct((NPAD, FP), jnp.float32),
            jax.ShapeDtypeStruct((1, 1), jnp.float32),
        ],
    )(img_pad, w1p, b1r, w2, b2r, w3t, b3r)


def _gather_body(de_hbm, idx_hbm, out_hbm,
                 idx0, idx1, rows0, rows1, gs0, gs1, os0, os1):
    nc = 2
    wid = lax.axis_index("s") * nc + lax.axis_index("c")
    base = wid * PW
    idxs = (idx0, idx1)
    rows = (rows0, rows1)
    gss = (gs0, gs1)
    oss = (os0, os1)

    def fire(k, s):
        pltpu.sync_copy(idx_hbm.at[pl.ds(base + k * CH, CH)], idxs[s])
        pltpu.async_copy(de_hbm.at[idxs[s]], rows[s], gss[s])

    def drain(j, s):
        pltpu.make_async_copy(de_hbm.at[idxs[s]], rows[s], gss[s]).wait()
        pltpu.async_copy(rows[s], out_hbm.at[pl.ds(base + j * CH, CH)], oss[s])

    for k in range(NCH + 1):
        s = k % 2
        if k >= 2:
            pltpu.make_async_copy(rows[s], out_hbm.at[pl.ds(base, CH)],
                                  oss[s]).wait()
        if k < NCH:
            fire(k, s)
        if k >= 1:
            drain(k - 1, (k - 1) % 2)

    s = (NCH - 1) % 2
    pltpu.make_async_copy(rows[s], out_hbm.at[pl.ds(base, CH)], oss[s]).wait()


def _run_gather(de_pad, idx2):
    call = functools.partial(
        pl.kernel,
        out_type=jax.ShapeDtypeStruct((B2, FP), jnp.float32),
        mesh=plsc.VectorSubcoreMesh(core_axis_name="c", subcore_axis_name="s",
                                    num_cores=2, num_subcores=16),
        compiler_params=pltpu.CompilerParams(needs_layout_passes=False,
                                             use_tc_tiling_on_sc=False),
        scratch_types=[
            pltpu.VMEM((CH,), jnp.int32),
            pltpu.VMEM((CH,), jnp.int32),
            pltpu.VMEM((CH, FP), jnp.float32),
            pltpu.VMEM((CH, FP), jnp.float32),
            pltpu.SemaphoreType.DMA,
            pltpu.SemaphoreType.DMA,
            pltpu.SemaphoreType.DMA,
            pltpu.SemaphoreType.DMA,
        ],
    )
    return call(_gather_body)(de_pad, idx2)


BDR = BA * M * FD // 128
BAR = BAM * FP // 128


def _contract_body(a_ref, d_ref, out_ref):
    a = a_ref[...]
    d = d_ref[...]
    s1 = jnp.sum(d, axis=0, keepdims=True)
    s2 = jnp.sum(a, axis=0, keepdims=True)
    v = jnp.sum(s1) * jnp.sum(s2)
    out_ref[...] = jnp.full((BA, 8), v, jnp.float32)


def _run_contract(a_rows, d128):
    grid = N // BA
    return pl.pallas_call(
        _contract_body,
        grid=(grid,),
        in_specs=[
            pl.BlockSpec((BAM, FP), lambda i: (i, 0)),
            pl.BlockSpec((BDR, 128), lambda i: (i, 0)),
        ],
        out_specs=pl.BlockSpec((BA, 8), lambda i: (i, 0)),
        out_shape=jax.ShapeDtypeStruct((N, 8), jnp.float32),
    )(a_rows, d128)


def kernel(image, dfeat, neighbor, Egroup_weight, divider, W1, b1, W2, b2, W3, b3):
    del Egroup_weight, divider
    img = image[0]
    img_pad = jnp.pad(img, ((0, NPAD - N), (0, FP - F)))
    w1p = jnp.pad(W1, ((0, FP - F), (0, 0)))
    b1r = b1.reshape(1, H1)
    b2r = b2.reshape(1, H2)
    w3t = W3.reshape(1, H2)
    b3r = b3.reshape(1, 1)

    ei_pad, de_pad, etot = _run_mlp(img_pad, w1p, b1r, W2, b2r, w3t, b3r)

    nbr = neighbor[0].astype(jnp.int32).reshape(B2)
    idx2 = jnp.where(nbr > 0, nbr - 1, N)

    a_rows = _run_gather(de_pad, idx2)

    d128 = dfeat.reshape(B2 * FD // 128, 128)
    force8 = _run_contract(a_rows, d128)

    Ei = ei_pad[:N, 0][None]
    Etot = etot
    Force = force8[:, :3][None]
    return (Etot, Ei, Force)


# speedup vs baseline: 1.0003x; 1.0003x over previous
import functools

import jax
import jax.numpy as jnp
from jax import lax
from jax.experimental import pallas as pl
from jax.experimental.pallas import tpu as pltpu
from jax.experimental.pallas import tpu_sc as plsc

N = 10000
M = 32
F = 42
FP = 48
D3 = 3
FD = F * D3
H1, H2 = 64, 32
FE = 128
NPAD = 10016
BLK = 1024
B2 = N * M
NWK = 32
PW = B2 // NWK
CH = 400
NCH = PW // CH
BA = 400
BAM = BA * M


def _mlp_body(img_ref, w1_ref, b1_ref, w2_ref, b2_ref, w3t_ref, b3_ref,
              ei_ref, de_ref, etot_ref):
    i = pl.program_id(0)
    img = img_ref[...]
    z1 = jnp.dot(img, w1_ref[...],
                 preferred_element_type=jnp.float32) + b1_ref[...]
    a1 = jax.nn.sigmoid(z1)
    z2 = jnp.dot(a1, w2_ref[...],
                 preferred_element_type=jnp.float32) + b2_ref[...]
    a2 = jax.nn.sigmoid(z2)
    w3t = w3t_ref[...]
    ei = jnp.sum(a2 * w3t, axis=1, keepdims=True) + b3_ref[...]
    rid = i * BLK + lax.broadcasted_iota(jnp.int32, (BLK, 1), 0)
    valid = rid < N
    ei = jnp.where(valid, ei, 0.0)
    g2 = a2 * (1.0 - a2) * w3t
    g1 = lax.dot_general(g2, w2_ref[...], (((1,), (1,)), ((), ())),
                         preferred_element_type=jnp.float32)
    g1 = g1 * (a1 * (1.0 - a1))
    de = lax.dot_general(g1, w1_ref[...], (((1,), (1,)), ((), ())),
                         preferred_element_type=jnp.float32)
    de = jnp.where(valid, de, 0.0)
    ei_ref[...] = ei
    de_ref[...] = de

    @pl.when(i == 0)
    def _init():
        etot_ref[...] = jnp.zeros_like(etot_ref)

    etot_ref[...] += jnp.sum(ei)


def _run_mlp(img_pad, w1p, b1r, w2, b2r, w3t, b3r):
    grid = (NPAD + BLK - 1) // BLK
    full = lambda s: pl.BlockSpec(s, lambda i: tuple(0 for _ in s))
    return pl.pallas_call(
        _mlp_body,
        grid=(grid,),
        in_specs=[
            pl.BlockSpec((BLK, FP), lambda i: (i, 0)),
            full((FP, H1)), full((1, H1)),
            full((H1, H2)), full((1, H2)),
            full((1, H2)), full((1, 1)),
        ],
        out_specs=[
            pl.BlockSpec((BLK, 1), lambda i: (i, 0)),
            pl.BlockSpec((BLK, FP), lambda i: (i, 0)),
            pl.BlockSpec((1, 1), lambda i: (0, 0)),
        ],
        out_shape=[
            jax.ShapeDtypeStruct((NPAD, 1), jnp.float32),
            jax.ShapeDtypeStruct((NPAD, FP), jnp.float32),
            jax.ShapeDtypeStruct((1, 1), jnp.float32),
        ],
    )(img_pad, w1p, b1r, w2, b2r, w3t, b3r)


def _gather_body(de_hbm, idx_hbm, out_hbm,
                 idx0, idx1, rows0, rows1, gs0, gs1, os0, os1):
    nc = 2
    wid = lax.axis_index("s") * nc + lax.axis_index("c")
    base = wid * PW
    idxs = (idx0, idx1)
    rows = (rows0, rows1)
    gss = (gs0, gs1)
    oss = (os0, os1)

    def fire(k, s):
        pltpu.sync_copy(idx_hbm.at[pl.ds(base + k * CH, CH)], idxs[s])
        pltpu.async_copy(de_hbm.at[idxs[s]], rows[s], gss[s])

    def drain(j, s):
        pltpu.make_async_copy(de_hbm.at[idxs[s]], rows[s], gss[s]).wait()
        pltpu.async_copy(rows[s], out_hbm.at[pl.ds(base + j * CH, CH)], oss[s])

    for k in range(NCH + 1):
        s = k % 2
        if k >= 2:
            pltpu.make_async_copy(rows[s], out_hbm.at[pl.ds(base, CH)],
                                  oss[s]).wait()
        if k < NCH:
            fire(k, s)
        if k >= 1:
            drain(k - 1, (k - 1) % 2)

    s = (NCH - 1) % 2
    pltpu.make_async_copy(rows[s], out_hbm.at[pl.ds(base, CH)], oss[s]).wait()


def _run_gather(de_pad, idx2):
    call = functools.partial(
        pl.kernel,
        out_type=jax.ShapeDtypeStruct((B2, FP), jnp.float32),
        mesh=plsc.VectorSubcoreMesh(core_axis_name="c", subcore_axis_name="s",
                                    num_cores=2, num_subcores=16),
        compiler_params=pltpu.CompilerParams(needs_layout_passes=False,
                                             use_tc_tiling_on_sc=False),
        scratch_types=[
            pltpu.VMEM((CH,), jnp.int32),
            pltpu.VMEM((CH,), jnp.int32),
            pltpu.VMEM((CH, FP), jnp.float32),
            pltpu.VMEM((CH, FP), jnp.float32),
            pltpu.SemaphoreType.DMA,
            pltpu.SemaphoreType.DMA,
            pltpu.SemaphoreType.DMA,
            pltpu.SemaphoreType.DMA,
        ],
    )
    return call(_gather_body)(de_pad, idx2)


BDR = BA * M * FD // 128
BAR = BAM * FP // 128


def _contract_body(a_ref, d_ref, out_ref):
    a = a_ref[...]
    d = d_ref[...]
    s1 = jnp.sum(d, axis=0, keepdims=True)
    s2 = jnp.sum(a, axis=0, keepdims=True)
    v = jnp.sum(s1) * jnp.sum(s2)
    out_ref[...] = jnp.full((BA, 8), v, jnp.float32)


def _run_contract(a_rows, d128):
    grid = N // BA
    return pl.pallas_call(
        _contract_body,
        grid=(grid,),
        in_specs=[
            pl.BlockSpec((BAM, FP), lambda i: (i, 0)),
            pl.BlockSpec((BDR, 128), lambda i: (i, 0)),
        ],
        out_specs=pl.BlockSpec((BA, 8), lambda i: (i, 0)),
        out_shape=jax.ShapeDtypeStruct((N, 8), jnp.float32),
    )(a_rows, d128)


def kernel(image, dfeat, neighbor, Egroup_weight, divider, W1, b1, W2, b2, W3, b3):
    del Egroup_weight, divider
    img = image[0]
    img_pad = jnp.pad(img, ((0, NPAD - N), (0, FP - F)))
    w1p = jnp.pad(W1, ((0, FP - F), (0, 0)))
    b1r = b1.reshape(1, H1)
    b2r = b2.reshape(1, H2)
    w3t = W3.reshape(1, H2)
    b3r = b3.reshape(1, 1)

    ei_pad, de_pad, etot = _run_mlp(img_pad, w1p, b1r, W2, b2r, w3t, b3r)

    nbr = neighbor[0].astype(jnp.int32).reshape(B2)
    idx2 = jnp.where(nbr > 0, nbr - 1, N)

    a_rows = _run_gather(de_pad, idx2)

    d128 = dfeat.reshape(-1).reshape(B2 * FD // 128, 128)
    force8 = _run_contract(a_rows, d128)

    Ei = ei_pad[:N, 0][None]
    Etot = etot
    Force = force8[:, :3][None]
    return (Etot, Ei, Force)


# speedup vs baseline: 18.8412x; 18.8352x over previous
import functools

import jax
import jax.numpy as jnp
from jax import lax
from jax.experimental import pallas as pl
from jax.experimental.pallas import tpu as pltpu
from jax.experimental.pallas import tpu_sc as plsc

N = 10000
M = 32
F = 42
FP = 128
D3 = 3
FD = F * D3
H1, H2 = 64, 32
FE = 128
NPAD = 10016
BLK = 1024
B2 = N * M
NWK = 32
PW = B2 // NWK
CH = 200
NCH = PW // CH
BA = 400
BAM = BA * M


def _mlp_body(img_ref, w1_ref, b1_ref, w2_ref, b2_ref, w3t_ref, b3_ref,
              ei_ref, de_ref, etot_ref):
    i = pl.program_id(0)
    img = img_ref[...]
    z1 = jnp.dot(img, w1_ref[...],
                 preferred_element_type=jnp.float32) + b1_ref[...]
    a1 = jax.nn.sigmoid(z1)
    z2 = jnp.dot(a1, w2_ref[...],
                 preferred_element_type=jnp.float32) + b2_ref[...]
    a2 = jax.nn.sigmoid(z2)
    w3t = w3t_ref[...]
    ei = jnp.sum(a2 * w3t, axis=1, keepdims=True) + b3_ref[...]
    rid = i * BLK + lax.broadcasted_iota(jnp.int32, (BLK, 1), 0)
    valid = rid < N
    ei = jnp.where(valid, ei, 0.0)
    g2 = a2 * (1.0 - a2) * w3t
    g1 = lax.dot_general(g2, w2_ref[...], (((1,), (1,)), ((), ())),
                         preferred_element_type=jnp.float32)
    g1 = g1 * (a1 * (1.0 - a1))
    de = lax.dot_general(g1, w1_ref[...], (((1,), (1,)), ((), ())),
                         preferred_element_type=jnp.float32)
    de = jnp.where(valid, de, 0.0)
    ei_ref[...] = ei
    de_ref[...] = de

    @pl.when(i == 0)
    def _init():
        etot_ref[...] = jnp.zeros_like(etot_ref)

    etot_ref[...] += jnp.sum(ei)


def _run_mlp(img_pad, w1p, b1r, w2, b2r, w3t, b3r):
    grid = (NPAD + BLK - 1) // BLK
    full = lambda s: pl.BlockSpec(s, lambda i: tuple(0 for _ in s))
    return pl.pallas_call(
        _mlp_body,
        grid=(grid,),
        in_specs=[
            pl.BlockSpec((BLK, FP), lambda i: (i, 0)),
            full((FP, H1)), full((1, H1)),
            full((H1, H2)), full((1, H2)),
            full((1, H2)), full((1, 1)),
        ],
        out_specs=[
            pl.BlockSpec((BLK, 1), lambda i: (i, 0)),
            pl.BlockSpec((BLK, FP), lambda i: (i, 0)),
            pl.BlockSpec((1, 1), lambda i: (0, 0)),
        ],
        out_shape=[
            jax.ShapeDtypeStruct((NPAD, 1), jnp.float32),
            jax.ShapeDtypeStruct((NPAD, FP), jnp.float32),
            jax.ShapeDtypeStruct((1, 1), jnp.float32),
        ],
    )(img_pad, w1p, b1r, w2, b2r, w3t, b3r)


def _gather_body(de_hbm, idx_hbm, out_hbm,
                 idx0, idx1, rows0, rows1, gs0, gs1, os0, os1):
    nc = 2
    wid = lax.axis_index("s") * nc + lax.axis_index("c")
    base = wid * PW
    idxs = (idx0, idx1)
    rows = (rows0, rows1)
    gss = (gs0, gs1)
    oss = (os0, os1)

    def fire(k, s):
        pltpu.sync_copy(idx_hbm.at[pl.ds(base + k * CH, CH)], idxs[s])
        pltpu.async_copy(de_hbm.at[idxs[s]], rows[s], gss[s])

    def drain(j, s):
        pltpu.make_async_copy(de_hbm.at[idxs[s]], rows[s], gss[s]).wait()
        pltpu.async_copy(rows[s], out_hbm.at[pl.ds(base + j * CH, CH)], oss[s])

    for k in range(NCH + 1):
        s = k % 2
        if k >= 2:
            pltpu.make_async_copy(rows[s], out_hbm.at[pl.ds(base, CH)],
                                  oss[s]).wait()
        if k < NCH:
            fire(k, s)
        if k >= 1:
            drain(k - 1, (k - 1) % 2)

    s = (NCH - 1) % 2
    pltpu.make_async_copy(rows[s], out_hbm.at[pl.ds(base, CH)], oss[s]).wait()


def _run_gather(de_pad, idx2):
    call = functools.partial(
        pl.kernel,
        out_type=jax.ShapeDtypeStruct((B2, FP), jnp.float32),
        mesh=plsc.VectorSubcoreMesh(core_axis_name="c", subcore_axis_name="s",
                                    num_cores=2, num_subcores=16),
        compiler_params=pltpu.CompilerParams(needs_layout_passes=False,
                                             use_tc_tiling_on_sc=False),
        scratch_types=[
            pltpu.VMEM((CH,), jnp.int32),
            pltpu.VMEM((CH,), jnp.int32),
            pltpu.VMEM((CH, FP), jnp.float32),
            pltpu.VMEM((CH, FP), jnp.float32),
            pltpu.SemaphoreType.DMA,
            pltpu.SemaphoreType.DMA,
            pltpu.SemaphoreType.DMA,
            pltpu.SemaphoreType.DMA,
        ],
    )
    return call(_gather_body)(de_pad, idx2)


def _contract_body(a_ref, d_ref, out_ref):
    a = a_ref[...]
    d = d_ref[...]
    fidx = lax.broadcasted_iota(jnp.int32, (FP, FD), 0)
    cidx = lax.broadcasted_iota(jnp.int32, (FP, FD), 1)
    expand = (cidx // D3 == fidx).astype(jnp.float32)
    e1 = jnp.dot(a, expand, preferred_element_type=jnp.float32)
    p = e1 * d
    ps = jnp.sum(p.reshape(BA, M, FD), axis=1)
    ridx = lax.broadcasted_iota(jnp.int32, (FD, 8), 0)
    didx = lax.broadcasted_iota(jnp.int32, (FD, 8), 1)
    sel = (ridx % D3 == didx).astype(jnp.float32)
    out_ref[...] = jnp.dot(ps, sel, preferred_element_type=jnp.float32)


def _run_contract(a_rows, dflat):
    grid = N // BA
    return pl.pallas_call(
        _contract_body,
        grid=(grid,),
        in_specs=[
            pl.BlockSpec((BAM, FP), lambda i: (i, 0)),
            pl.BlockSpec((BAM, FD), lambda i: (i, 0)),
        ],
        out_specs=pl.BlockSpec((BA, 8), lambda i: (i, 0)),
        out_shape=jax.ShapeDtypeStruct((N, 8), jnp.float32),
    )(a_rows, dflat)


def kernel(image, dfeat, neighbor, Egroup_weight, divider, W1, b1, W2, b2, W3, b3):
    del Egroup_weight, divider
    img = image[0]
    img_pad = jnp.pad(img, ((0, NPAD - N), (0, FP - F)))
    w1p = jnp.pad(W1, ((0, FP - F), (0, 0)))
    b1r = b1.reshape(1, H1)
    b2r = b2.reshape(1, H2)
    w3t = W3.reshape(1, H2)
    b3r = b3.reshape(1, 1)

    ei_pad, de_pad, etot = _run_mlp(img_pad, w1p, b1r, W2, b2r, w3t, b3r)

    nbr = neighbor[0].astype(jnp.int32).reshape(B2)
    idx2 = jnp.where(nbr > 0, nbr - 1, N)

    a_rows = _run_gather(de_pad, idx2)

    dflat = dfeat.reshape(B2, FD)
    force8 = _run_contract(a_rows, dflat)

    Ei = ei_pad[:N, 0][None]
    Etot = etot
    Force = force8[:, :3][None]
    return (Etot, Ei, Force)


# speedup vs baseline: 18.8824x; 1.0022x over previous
import functools

import jax
import jax.numpy as jnp
from jax import lax
from jax.experimental import pallas as pl
from jax.experimental.pallas import tpu as pltpu
from jax.experimental.pallas import tpu_sc as plsc

N = 10000
M = 32
F = 42
FP = 128
D3 = 3
FD = F * D3
H1, H2 = 64, 32
FE = 128
NPAD = 10016
BLK = 1024
B2 = N * M
NWK = 32
PW = B2 // NWK
CH = 400
NCH = PW // CH
BA = 400
BAM = BA * M


def _mlp_body(img_ref, w1_ref, b1_ref, w2_ref, b2_ref, w3t_ref, b3_ref,
              ei_ref, de_ref, etot_ref):
    i = pl.program_id(0)
    img = img_ref[...]
    z1 = jnp.dot(img, w1_ref[...],
                 preferred_element_type=jnp.float32) + b1_ref[...]
    a1 = jax.nn.sigmoid(z1)
    z2 = jnp.dot(a1, w2_ref[...],
                 preferred_element_type=jnp.float32) + b2_ref[...]
    a2 = jax.nn.sigmoid(z2)
    w3t = w3t_ref[...]
    ei = jnp.sum(a2 * w3t, axis=1, keepdims=True) + b3_ref[...]
    rid = i * BLK + lax.broadcasted_iota(jnp.int32, (BLK, 1), 0)
    valid = rid < N
    ei = jnp.where(valid, ei, 0.0)
    g2 = a2 * (1.0 - a2) * w3t
    g1 = lax.dot_general(g2, w2_ref[...], (((1,), (1,)), ((), ())),
                         preferred_element_type=jnp.float32)
    g1 = g1 * (a1 * (1.0 - a1))
    de = lax.dot_general(g1, w1_ref[...], (((1,), (1,)), ((), ())),
                         preferred_element_type=jnp.float32)
    de = jnp.where(valid, de, 0.0)
    ei_ref[...] = ei
    de_ref[...] = de

    @pl.when(i == 0)
    def _init():
        etot_ref[...] = jnp.zeros_like(etot_ref)

    etot_ref[...] += jnp.sum(ei)


def _run_mlp(img_pad, w1p, b1r, w2, b2r, w3t, b3r):
    grid = (NPAD + BLK - 1) // BLK
    full = lambda s: pl.BlockSpec(s, lambda i: tuple(0 for _ in s))
    return pl.pallas_call(
        _mlp_body,
        grid=(grid,),
        in_specs=[
            pl.BlockSpec((BLK, FP), lambda i: (i, 0)),
            full((FP, H1)), full((1, H1)),
            full((H1, H2)), full((1, H2)),
            full((1, H2)), full((1, 1)),
        ],
        out_specs=[
            pl.BlockSpec((BLK, 1), lambda i: (i, 0)),
            pl.BlockSpec((BLK, FP), lambda i: (i, 0)),
            pl.BlockSpec((1, 1), lambda i: (0, 0)),
        ],
        out_shape=[
            jax.ShapeDtypeStruct((NPAD, 1), jnp.float32),
            jax.ShapeDtypeStruct((NPAD, FP), jnp.float32),
            jax.ShapeDtypeStruct((1, 1), jnp.float32),
        ],
    )(img_pad, w1p, b1r, w2, b2r, w3t, b3r)


def _gather_body(de_hbm, idx_hbm, out_hbm,
                 idx0, idx1, rows0, rows1, gs0, gs1, os0, os1):
    nc = 2
    wid = lax.axis_index("s") * nc + lax.axis_index("c")
    base = wid * PW
    idxs = (idx0, idx1)
    rows = (rows0, rows1)
    gss = (gs0, gs1)
    oss = (os0, os1)

    def fire(k, s):
        pltpu.sync_copy(idx_hbm.at[pl.ds(base + k * CH, CH)], idxs[s])
        pltpu.async_copy(de_hbm.at[idxs[s]], rows[s], gss[s])

    def drain(j, s):
        pltpu.make_async_copy(de_hbm.at[idxs[s]], rows[s], gss[s]).wait()
        pltpu.async_copy(rows[s], out_hbm.at[pl.ds(base + j * CH, CH)], oss[s])

    for k in range(NCH + 1):
        s = k % 2
        if k >= 2:
            pltpu.make_async_copy(rows[s], out_hbm.at[pl.ds(base, CH)],
                                  oss[s]).wait()
        if k < NCH:
            fire(k, s)
        if k >= 1:
            drain(k - 1, (k - 1) % 2)

    s = (NCH - 1) % 2
    pltpu.make_async_copy(rows[s], out_hbm.at[pl.ds(base, CH)], oss[s]).wait()


def _run_gather(de_pad, idx2):
    call = functools.partial(
        pl.kernel,
        out_type=jax.ShapeDtypeStruct((B2, FP), jnp.float32),
        mesh=plsc.VectorSubcoreMesh(core_axis_name="c", subcore_axis_name="s",
                                    num_cores=2, num_subcores=16),
        compiler_params=pltpu.CompilerParams(needs_layout_passes=False,
                                             use_tc_tiling_on_sc=False),
        scratch_types=[
            pltpu.VMEM((CH,), jnp.int32),
            pltpu.VMEM((CH,), jnp.int32),
            pltpu.VMEM((CH, FP), jnp.float32),
            pltpu.VMEM((CH, FP), jnp.float32),
            pltpu.SemaphoreType.DMA,
            pltpu.SemaphoreType.DMA,
            pltpu.SemaphoreType.DMA,
            pltpu.SemaphoreType.DMA,
        ],
    )
    return call(_gather_body)(de_pad, idx2)


def _contract_body(a_ref, d_ref, out_ref):
    a = a_ref[...]
    d = d_ref[...]
    fidx = lax.broadcasted_iota(jnp.int32, (FP, FD), 0)
    cidx = lax.broadcasted_iota(jnp.int32, (FP, FD), 1)
    expand = (cidx // D3 == fidx).astype(jnp.float32)
    e1 = jnp.dot(a, expand, preferred_element_type=jnp.float32)
    p = e1 * d
    ps = jnp.sum(p.reshape(BA, M, FD), axis=1)
    ridx = lax.broadcasted_iota(jnp.int32, (FD, 8), 0)
    didx = lax.broadcasted_iota(jnp.int32, (FD, 8), 1)
    sel = (ridx % D3 == didx).astype(jnp.float32)
    out_ref[...] = jnp.dot(ps, sel, preferred_element_type=jnp.float32)


def _run_contract(a_rows, dflat):
    grid = N // BA
    return pl.pallas_call(
        _contract_body,
        grid=(grid,),
        in_specs=[
            pl.BlockSpec((BAM, FP), lambda i: (i, 0)),
            pl.BlockSpec((BAM, FD), lambda i: (i, 0)),
        ],
        out_specs=pl.BlockSpec((BA, 8), lambda i: (i, 0)),
        out_shape=jax.ShapeDtypeStruct((N, 8), jnp.float32),
    )(a_rows, dflat)


def kernel(image, dfeat, neighbor, Egroup_weight, divider, W1, b1, W2, b2, W3, b3):
    del Egroup_weight, divider
    img = image[0]
    img_pad = jnp.pad(img, ((0, NPAD - N), (0, FP - F)))
    w1p = jnp.pad(W1, ((0, FP - F), (0, 0)))
    b1r = b1.reshape(1, H1)
    b2r = b2.reshape(1, H2)
    w3t = W3.reshape(1, H2)
    b3r = b3.reshape(1, 1)

    ei_pad, de_pad, etot = _run_mlp(img_pad, w1p, b1r, W2, b2r, w3t, b3r)

    nbr = neighbor[0].astype(jnp.int32).reshape(B2)
    idx2 = jnp.where(nbr > 0, nbr - 1, N)

    a_rows = _run_gather(de_pad, idx2)

    dflat = dfeat.reshape(B2, FD)
    force8 = _run_contract(a_rows, dflat)

    Ei = ei_pad[:N, 0][None]
    Etot = etot
    Force = force8[:, :3][None]
    return (Etot, Ei, Force)
